# baseline (speedup 1.0000x reference)
import functools

import jax
import jax.numpy as jnp
from jax.experimental import pallas as pl
from jax.experimental.pallas import tpu as pltpu

_NUM_CLASSES = 10


def _cparams():
    return pltpu.CompilerParams(dimension_semantics=("parallel",),
                                vmem_limit_bytes=64 * 1024 * 1024)


def _conv_bn_relu(x, w_ref, b_ref):
    G, H, W, C = x.shape
    Co = w_ref.shape[-1]
    zw = jnp.zeros((G, H, 1, C), x.dtype)
    xw = jnp.concatenate([zw, x, zw], axis=2)
    zh = jnp.zeros((G, 1, W + 2, C), x.dtype)
    xp = jnp.concatenate([zh, xw, zh], axis=1)
    slab = jnp.concatenate(
        [xp[:, :, 0:W, :], xp[:, :, 1:W + 1, :], xp[:, :, 2:W + 2, :]],
        axis=3)
    acc = jnp.dot(slab[:, 0:H].reshape(G * H * W, 3 * C), w_ref[0],
                  preferred_element_type=jnp.float32)
    acc += jnp.dot(slab[:, 1:H + 1].reshape(G * H * W, 3 * C), w_ref[1],
                   preferred_element_type=jnp.float32)
    acc += jnp.dot(slab[:, 2:H + 2].reshape(G * H * W, 3 * C), w_ref[2],
                   preferred_element_type=jnp.float32)
    y = jnp.maximum(acc + b_ref[...], 0.0).astype(jnp.bfloat16)
    return y.reshape(G, H, W, Co)


def _maxpool2x2(x):
    G, H, W, C = x.shape
    x5 = x.reshape(G, H // 2, 2, W, C)
    h = jnp.maximum(x5[:, :, 0], x5[:, :, 1])
    pieces = [jnp.maximum(h[:, :, 2 * k:2 * k + 1, :],
                          h[:, :, 2 * k + 1:2 * k + 2, :])
              for k in range(W // 2)]
    return jnp.concatenate(pieces, axis=2)


def _stage_body(*refs, segs, stem, fc):
    x_ref, o_ref = refs[0], refs[-1]
    params = refs[1:-1]
    i = 0
    if stem:
        s = x_ref[...]
        G, H, W, C = s.shape
        w, b = params[0], params[1]
        i = 2
        acc = jnp.dot(s.reshape(G * H * W, C), w[...],
                      preferred_element_type=jnp.float32)
        x = jnp.maximum(acc + b[...], 0.0).astype(jnp.bfloat16)
        x = x.reshape(G, H, W, w.shape[-1])
    else:
        x = x_ref[...]
    first = stem
    for nconv in segs:
        for _ in range(nconv - (1 if first else 0)):
            x = _conv_bn_relu(x, params[i], params[i + 1])
            i += 2
        first = False
        x = _maxpool2x2(x)
    if fc:
        wc, bc = params[i], params[i + 1]
        flat = x.reshape(x.shape[0], x.shape[-1])
        o_ref[...] = (jnp.dot(flat, wc[...],
                              preferred_element_type=jnp.float32) + bc[...])
    else:
        o_ref[...] = x


def _const_map(i, nd):
    return (0,) * nd


def _lead_map(i, nd):
    return (i,) + (0,) * (nd - 1)


def _run_stage(x, params, group, segs, out_tail, stem=False, fc=False):
    N = x.shape[0]
    G = group
    while N % G:
        G //= 2
    in_specs = [pl.BlockSpec((G,) + x.shape[1:],
                             functools.partial(_lead_map, nd=x.ndim))]
    for p in params:
        in_specs.append(pl.BlockSpec(p.shape,
                                     functools.partial(_const_map, nd=p.ndim)))
    odtype = jnp.float32 if fc else jnp.bfloat16
    return pl.pallas_call(
        functools.partial(_stage_body, segs=segs, stem=stem, fc=fc),
        out_shape=jax.ShapeDtypeStruct((N,) + out_tail, odtype),
        grid=(N // G,),
        in_specs=in_specs,
        out_specs=pl.BlockSpec((G,) + out_tail,
                               functools.partial(_lead_map, nd=1 + len(out_tail))),
        compiler_params=_cparams(),
    )(x, *params)


def _stem_slab(x_nchw):
    xh = jnp.transpose(x_nchw, (0, 2, 3, 1)).astype(jnp.bfloat16)
    N, H, W, _ = xh.shape
    xp = jnp.pad(xh, ((0, 0), (1, 1), (1, 1), (0, 0)))
    pieces = [xp[:, dy:dy + H, dx:dx + W, :]
              for dy in range(3) for dx in range(3)]
    slab = jnp.concatenate(pieces, axis=-1)
    return jnp.pad(slab, ((0, 0), (0, 0), (0, 0), (0, 5)))


def _stem_weight(w3):
    w = w3.reshape(3, 3, 32, -1)[:, :, :3, :].reshape(27, -1)
    return jnp.pad(w, ((0, 5), (0, 0)))


def kernel(x, conv_w_0, conv_shift_0, conv_w_1, conv_shift_1,
           conv_w_2, conv_shift_2, conv_w_3, conv_shift_3,
           conv_w_4, conv_shift_4, conv_w_5, conv_shift_5,
           conv_w_6, conv_shift_6, conv_w_7, conv_shift_7,
           conv_w_8, conv_shift_8, conv_w_9, conv_shift_9,
           conv_w_10, conv_shift_10, conv_w_11, conv_shift_11,
           conv_w_12, conv_shift_12, cls_w, cls_b):
    cs = [s.reshape(1, -1) for s in (
        conv_shift_0, conv_shift_1, conv_shift_2, conv_shift_3, conv_shift_4,
        conv_shift_5, conv_shift_6, conv_shift_7, conv_shift_8, conv_shift_9,
        conv_shift_10, conv_shift_11, conv_shift_12)]

    s0 = _stem_slab(x)
    w0 = _stem_weight(conv_w_0)

    p2 = _run_stage(s0, (w0, cs[0], conv_w_1, cs[1], conv_w_2, cs[2],
                         conv_w_3, cs[3]),
                    group=4, segs=(2, 2), out_tail=(8, 8, 128), stem=True)
    p4 = _run_stage(p2, (conv_w_4, cs[4], conv_w_5, cs[5], conv_w_6, cs[6],
                         conv_w_7, cs[7], conv_w_8, cs[8], conv_w_9, cs[9]),
                    group=32, segs=(3, 3), out_tail=(2, 2, 512))
    logits = _run_stage(
        p4, (conv_w_10, cs[10], conv_w_11, cs[11], conv_w_12, cs[12],
             cls_w, cls_b.reshape(1, -1)),
        group=64, segs=(3,), out_tail=(cls_w.shape[1],), fc=True)
    return logits[:, :_NUM_CLASSES]


# speedup vs baseline: 1.0487x; 1.0487x over previous
import functools

import jax
import jax.numpy as jnp
from jax.experimental import pallas as pl
from jax.experimental.pallas import tpu as pltpu

_NUM_CLASSES = 10


def _cparams():
    return pltpu.CompilerParams(dimension_semantics=("parallel",),
                                vmem_limit_bytes=64 * 1024 * 1024)


def _conv_bn_relu(x, w_ref, b_ref):
    G, H, W, C = x.shape
    Co = w_ref.shape[-1]
    zw = jnp.zeros((G, H, 1, C), x.dtype)
    xw = jnp.concatenate([zw, x, zw], axis=2)
    zh = jnp.zeros((G, 1, W + 2, C), x.dtype)
    xp = jnp.concatenate([zh, xw, zh], axis=1)
    slab = jnp.concatenate(
        [xp[:, :, 0:W, :], xp[:, :, 1:W + 1, :], xp[:, :, 2:W + 2, :]],
        axis=3)
    acc = jnp.dot(slab[:, 0:H].reshape(G * H * W, 3 * C), w_ref[0],
                  preferred_element_type=jnp.float32)
    acc += jnp.dot(slab[:, 1:H + 1].reshape(G * H * W, 3 * C), w_ref[1],
                   preferred_element_type=jnp.float32)
    acc += jnp.dot(slab[:, 2:H + 2].reshape(G * H * W, 3 * C), w_ref[2],
                   preferred_element_type=jnp.float32)
    y = jnp.maximum(acc + b_ref[...], 0.0).astype(jnp.bfloat16)
    return y.reshape(G, H, W, Co)


def _maxpool2x2(x):
    G, H, W, C = x.shape
    x5 = x.reshape(G, H // 2, 2, W, C)
    h = jnp.maximum(x5[:, :, 0], x5[:, :, 1])
    pieces = [jnp.maximum(h[:, :, 2 * k:2 * k + 1, :],
                          h[:, :, 2 * k + 1:2 * k + 2, :])
              for k in range(W // 2)]
    return jnp.concatenate(pieces, axis=2)


def _stage_body(*refs, segs, stem, fc):
    x_ref, o_ref = refs[0], refs[-1]
    params = refs[1:-1]
    i = 0
    if stem:
        s = x_ref[...]
        G, H, W, C = s.shape
        w, b = params[0], params[1]
        i = 2
        acc = jnp.dot(s.reshape(G * H * W, C), w[...],
                      preferred_element_type=jnp.float32)
        x = jnp.maximum(acc + b[...], 0.0).astype(jnp.bfloat16)
        x = x.reshape(G, H, W, w.shape[-1])
    else:
        x = x_ref[...]
    first = stem
    for nconv in segs:
        for _ in range(nconv - (1 if first else 0)):
            x = _conv_bn_relu(x, params[i], params[i + 1])
            i += 2
        first = False
        x = _maxpool2x2(x)
    if fc:
        wc, bc = params[i], params[i + 1]
        flat = x.reshape(x.shape[0], x.shape[-1])
        o_ref[...] = (jnp.dot(flat, wc[...],
                              preferred_element_type=jnp.float32) + bc[...])
    else:
        o_ref[...] = x


def _const_map(i, nd):
    return (0,) * nd


def _lead_map(i, nd):
    return (i,) + (0,) * (nd - 1)


def _run_stage(x, params, group, segs, out_tail, stem=False, fc=False):
    N = x.shape[0]
    G = group
    while N % G:
        G //= 2
    in_specs = [pl.BlockSpec((G,) + x.shape[1:],
                             functools.partial(_lead_map, nd=x.ndim))]
    for p in params:
        in_specs.append(pl.BlockSpec(p.shape,
                                     functools.partial(_const_map, nd=p.ndim)))
    odtype = jnp.float32 if fc else jnp.bfloat16
    return pl.pallas_call(
        functools.partial(_stage_body, segs=segs, stem=stem, fc=fc),
        out_shape=jax.ShapeDtypeStruct((N,) + out_tail, odtype),
        grid=(N // G,),
        in_specs=in_specs,
        out_specs=pl.BlockSpec((G,) + out_tail,
                               functools.partial(_lead_map, nd=1 + len(out_tail))),
        compiler_params=_cparams(),
    )(x, *params)


def _stem_slab(x_nchw):
    xh = jnp.transpose(x_nchw, (0, 2, 3, 1)).astype(jnp.bfloat16)
    N, H, W, _ = xh.shape
    xp = jnp.pad(xh, ((0, 0), (1, 1), (1, 1), (0, 0)))
    pieces = [xp[:, dy:dy + H, dx:dx + W, :]
              for dy in range(3) for dx in range(3)]
    slab = jnp.concatenate(pieces, axis=-1)
    return jnp.pad(slab, ((0, 0), (0, 0), (0, 0), (0, 5)))


def _stem_weight(w3):
    w = w3.reshape(3, 3, 32, -1)[:, :, :3, :].reshape(27, -1)
    return jnp.pad(w, ((0, 5), (0, 0)))


def kernel(x, conv_w_0, conv_shift_0, conv_w_1, conv_shift_1,
           conv_w_2, conv_shift_2, conv_w_3, conv_shift_3,
           conv_w_4, conv_shift_4, conv_w_5, conv_shift_5,
           conv_w_6, conv_shift_6, conv_w_7, conv_shift_7,
           conv_w_8, conv_shift_8, conv_w_9, conv_shift_9,
           conv_w_10, conv_shift_10, conv_w_11, conv_shift_11,
           conv_w_12, conv_shift_12, cls_w, cls_b):
    cs = [s.reshape(1, -1) for s in (
        conv_shift_0, conv_shift_1, conv_shift_2, conv_shift_3, conv_shift_4,
        conv_shift_5, conv_shift_6, conv_shift_7, conv_shift_8, conv_shift_9,
        conv_shift_10, conv_shift_11, conv_shift_12)]

    s0 = _stem_slab(x)
    w0 = _stem_weight(conv_w_0)

    p1 = _run_stage(s0, (w0, cs[0], conv_w_1, cs[1]),
                    group=8, segs=(2,), out_tail=(16, 16, 128), stem=True)
    p2 = _run_stage(p1, (conv_w_2, cs[2], conv_w_3, cs[3]),
                    group=32, segs=(2,), out_tail=(8, 8, 128))
    p3 = _run_stage(p2, (conv_w_4, cs[4], conv_w_5, cs[5], conv_w_6, cs[6]),
                    group=32, segs=(3,), out_tail=(4, 4, 256))
    p4 = _run_stage(p3, (conv_w_7, cs[7], conv_w_8, cs[8], conv_w_9, cs[9]),
                    group=64, segs=(3,), out_tail=(2, 2, 512))
    logits = _run_stage(
        p4, (conv_w_10, cs[10], conv_w_11, cs[11], conv_w_12, cs[12],
             cls_w, cls_b.reshape(1, -1)),
        group=128, segs=(3,), out_tail=(cls_w.shape[1],), fc=True)
    return logits[:, :_NUM_CLASSES]
